# baseline (speedup 1.0000x reference)
import functools

import jax
import jax.numpy as jnp
from jax import lax
from jax.experimental import pallas as pl
from jax.experimental.pallas import tpu as pltpu
from jax.experimental.pallas import tpu_sc as plsc

_T = 2048
_D = 768
_I = 3072
_E = 8
_IB = 512
_NI = _I // _IB
_CH = 512
_MAXCH = 16
_NSLOT = 2 * _T
_PAD = _MAXCH * _CH


def _gelu(v):
    return 0.5 * v * (1.0 + lax.erf(v * (2.0 ** -0.5)))


def _router_body(x_ref, wr1_ref, wr2_ref,
                 sel_ref, ws_ref, w0_ref, w1_ref, any_ref):
    x = x_ref[...]
    h = _gelu(lax.dot_general(x, wr1_ref[...], (((1,), (1,)), ((), ())),
                              preferred_element_type=jnp.float32))
    logits = lax.dot_general(h, wr2_ref[...], (((1,), (1,)), ((), ())),
                             preferred_element_type=jnp.float32)
    m = jnp.max(logits, axis=-1, keepdims=True)
    ex = jnp.exp(logits - m)
    p = ex / jnp.sum(ex, axis=-1, keepdims=True)

    iota = lax.broadcasted_iota(jnp.int32, (_T, _E), 1)
    big = jnp.int32(_E + 1)
    m0 = jnp.max(p, axis=-1, keepdims=True)
    i0 = jnp.min(jnp.where(p == m0, iota, big), axis=-1, keepdims=True)
    p2 = jnp.where(iota == i0, -1.0, p)
    m1 = jnp.max(p2, axis=-1, keepdims=True)
    i1 = jnp.min(jnp.where(p2 == m1, iota, big), axis=-1, keepdims=True)

    sel_ref[...] = jnp.concatenate([i0, i1], axis=1)
    ws_ref[...] = jnp.concatenate([m0, m1], axis=1)
    w0_ref[...] = m0
    w1_ref[...] = m1
    oh0 = (iota == i0).astype(jnp.float32)
    oh1 = (iota == i1).astype(jnp.float32)
    any_ref[0:1, :] = jnp.max(oh0, axis=0, keepdims=True)
    any_ref[1:2, :] = jnp.max(oh1, axis=0, keepdims=True)


def _router(xf, Wr1, Wr2):
    return pl.pallas_call(
        _router_body,
        grid=(1,),
        in_specs=[
            pl.BlockSpec((_T, _D), lambda i: (0, 0)),
            pl.BlockSpec((_D // 2, _D), lambda i: (0, 0)),
            pl.BlockSpec((_E, _D // 2), lambda i: (0, 0)),
        ],
        out_specs=[
            pl.BlockSpec((_T, 2), lambda i: (0, 0)),
            pl.BlockSpec((_T, 2), lambda i: (0, 0)),
            pl.BlockSpec((_T, 1), lambda i: (0, 0)),
            pl.BlockSpec((_T, 1), lambda i: (0, 0)),
            pl.BlockSpec((2, _E), lambda i: (0, 0)),
        ],
        out_shape=[
            jax.ShapeDtypeStruct((_T, 2), jnp.int32),
            jax.ShapeDtypeStruct((_T, 2), jnp.float32),
            jax.ShapeDtypeStruct((_T, 1), jnp.float32),
            jax.ShapeDtypeStruct((_T, 1), jnp.float32),
            jax.ShapeDtypeStruct((2, _E), jnp.float32),
        ],
    )(xf, Wr1, Wr2)


_NTILE = 16
_SPT = _NSLOT // _NTILE
_NV = _SPT // 16


def _dispatch_body(sel_hbm, ws_hbm, perm_out, pw_out, inv_out, cex_out, val_out,
                   selv, wsv, cntv, callv, tokv, invv, auxv, zi, zf,
                   csh, psh, wsh, sem):
    cid = lax.axis_index("c")
    tid = lax.axis_index("s")
    iota = lax.iota(jnp.int32, 16)

    @pl.when(cid == 0)
    def _():
        base = tid * _SPT
        sbase = tid * (_PAD // _NTILE)
        pltpu.sync_copy(sel_hbm.at[pl.ds(base, _SPT)], selv)
        pltpu.sync_copy(ws_hbm.at[pl.ds(base, _SPT)], wsv)

        for v in range((_PAD // _NTILE) // 16):
            zi[pl.ds(v * 16, 16)] = jnp.zeros((16,), jnp.int32)
            zf[pl.ds(v * 16, 16)] = jnp.zeros((16,), jnp.float32)
        pltpu.sync_copy(zi, psh.at[pl.ds(sbase, _PAD // _NTILE)])
        pltpu.sync_copy(zf, wsh.at[pl.ds(sbase, _PAD // _NTILE)])

        counts = jnp.zeros((16,), jnp.int32)
        for v in range(_NV):
            sv = selv[pl.ds(v * 16, 16)]
            for e in range(_E):
                ind = jnp.where(sv == e, 1, 0)
                cnt = jnp.sum(ind)
                counts = counts + jnp.where(iota == e, cnt, 0)
        cntv[...] = counts
        pltpu.sync_copy(cntv, csh.at[tid])
        plsc.subcore_barrier()

        pltpu.sync_copy(csh, callv)
        totals = jnp.zeros((16,), jnp.int32)
        tprefix = jnp.zeros((16,), jnp.int32)
        for w in range(_NTILE):
            row = callv[w]
            totals = totals + row
            tprefix = tprefix + jnp.where(jnp.int32(w) < tid, row, 0)
        nch = (totals + (_CH - 1)) >> 9
        incl = plsc.cumsum(nch)
        excl = incl - nch
        basev = excl * _CH
        r_vec = basev + tprefix

        for v in range(_NV):
            sv = selv[pl.ds(v * 16, 16)]
            slot_v = base + v * 16 + iota
            tok_v = slot_v >> 1
            dst_v = jnp.zeros((16,), jnp.int32)
            for e in range(_E):
                m = sv == e
                ind = jnp.where(m, 1, 0)
                pre_rank = plsc.cumsum(ind) - ind
                re = jnp.sum(jnp.where(iota == e, r_vec, 0))
                dst_v = dst_v + jnp.where(m, re + pre_rank, 0)
                r_vec = r_vec + jnp.where(iota == e, jnp.sum(ind), 0)
            tokv[pl.ds(v * 16, 16)] = tok_v
            invv[pl.ds(v * 16, 16)] = dst_v & (_PAD - 1)
        pltpu.sync_copy(tokv, psh.at[invv])
        pltpu.sync_copy(wsv, wsh.at[invv])
        pltpu.sync_copy(invv, inv_out.at[pl.ds(base, _SPT)])
        plsc.subcore_barrier()
        pltpu.sync_copy(psh.at[pl.ds(sbase, _PAD // _NTILE)],
                        perm_out.at[pl.ds(sbase, _PAD // _NTILE)])
        pltpu.sync_copy(wsh.at[pl.ds(sbase, _PAD // _NTILE)],
                        pw_out.at[pl.ds(sbase, _PAD // _NTILE)])

        @pl.when(tid == 0)
        def _():
            total_chunks = jnp.sum(nch)
            cex = jnp.zeros((16,), jnp.int32)
            val = jnp.zeros((16,), jnp.int32)
            for e in range(_E):
                e_excl = jnp.sum(jnp.where(iota == e, excl, 0))
                e_incl = jnp.sum(jnp.where(iota == e, incl, 0))
                e_tot = jnp.sum(jnp.where(iota == e, totals, 0))
                in_r = (iota >= e_excl) & (iota < e_incl)
                cex = cex + jnp.where(in_r, e, 0)
                raw = e_tot - (iota - e_excl) * _CH
                raw = jnp.minimum(jnp.maximum(raw, 0), _CH)
                val = val + jnp.where(in_r, raw, 0)
            cex = jnp.where(iota < total_chunks, cex, _E - 1)
            auxv[...] = cex
            pltpu.sync_copy(auxv, cex_out)
            auxv[...] = val
            pltpu.sync_copy(auxv, val_out)


def _dispatch(sel_flat, ws_flat):
    f = functools.partial(
        pl.kernel,
        mesh=plsc.VectorSubcoreMesh(core_axis_name="c", subcore_axis_name="s"),
        compiler_params=pltpu.CompilerParams(needs_layout_passes=False),
        out_type=[
            jax.ShapeDtypeStruct((_PAD,), jnp.int32),
            jax.ShapeDtypeStruct((_PAD,), jnp.float32),
            jax.ShapeDtypeStruct((_NSLOT,), jnp.int32),
            jax.ShapeDtypeStruct((16,), jnp.int32),
            jax.ShapeDtypeStruct((16,), jnp.int32),
        ],
        scratch_types=[
            pltpu.VMEM((_SPT,), jnp.int32),
            pltpu.VMEM((_SPT,), jnp.float32),
            pltpu.VMEM((16,), jnp.int32),
            pltpu.VMEM((_NTILE, 16), jnp.int32),
            pltpu.VMEM((_SPT,), jnp.int32),
            pltpu.VMEM((_SPT,), jnp.int32),
            pltpu.VMEM((16,), jnp.int32),
            pltpu.VMEM((_PAD // _NTILE,), jnp.int32),
            pltpu.VMEM((_PAD // _NTILE,), jnp.float32),
            pltpu.VMEM_SHARED((_NTILE, 16), jnp.int32),
            pltpu.VMEM_SHARED((_PAD,), jnp.int32),
            pltpu.VMEM_SHARED((_PAD,), jnp.float32),
            pltpu.SemaphoreType.DMA,
        ],
    )
    return f(_dispatch_body)(sel_flat, ws_flat)


_NW = 32
_RPW = _PAD // _NW


def _gather_body(perm_hbm, x_hbm, xg_out, idxv, idx2, rows, sem):
    wid = lax.axis_index("s") * 2 + lax.axis_index("c")
    base = wid * _RPW
    pltpu.sync_copy(perm_hbm.at[pl.ds(base, _RPW)], idxv)
    for h in range(2):
        for v in range(_RPW // 32):
            idx2[pl.ds(v * 16, 16)] = (
                idxv[pl.ds(h * (_RPW // 2) + v * 16, 16)] & (_T - 1))
        pltpu.async_copy(x_hbm.at[idx2], rows, sem).wait()
        pltpu.sync_copy(rows, xg_out.at[pl.ds(base + h * (_RPW // 2), _RPW // 2)])


def _gather(perm, xf):
    f = functools.partial(
        pl.kernel,
        mesh=plsc.VectorSubcoreMesh(core_axis_name="c", subcore_axis_name="s"),
        compiler_params=pltpu.CompilerParams(needs_layout_passes=False),
        out_type=jax.ShapeDtypeStruct((_PAD, _D), jnp.float32),
        scratch_types=[
            pltpu.VMEM((_RPW,), jnp.int32),
            pltpu.VMEM((_RPW // 2,), jnp.int32),
            pltpu.VMEM((_RPW // 2, _D), jnp.float32),
            pltpu.SemaphoreType.DMA,
        ],
    )
    return f(_gather_body)(perm, xf)


def _ffn_body(cex_ref, val_ref,
              xg_ref, pw_ref, w0_ref, w1_ref, any_ref, wi_ref, bi_ref, wo_ref,
              bo_ref, zg_ref, oinit_ref, const_scr):
    c = pl.program_id(0)
    i = pl.program_id(1)

    @pl.when(jnp.logical_and(c == 0, i == 0))
    def _():
        const_scr[...] = jnp.zeros((_E, _D), jnp.float32)
        oinit_ref[...] = jnp.zeros((_T, _D), jnp.float32)

    @pl.when(i == 0)
    def _():
        zg_ref[...] = jnp.zeros((_CH, _D), jnp.float32)

    xg = xg_ref[...]
    wi = wi_ref[0]
    wo = wo_ref[0]
    bi_row = bi_ref[0, 0]

    pre = lax.dot_general(xg, wi, (((1,), (1,)), ((), ())),
                          preferred_element_type=jnp.float32) + bi_row
    gb = _gelu(bi_row)
    act = _gelu(pre) - gb
    rowid = lax.broadcasted_iota(jnp.int32, (_CH, 1), 0)
    valc = val_ref[c]
    maskr = rowid < valc
    pwcol = jnp.where(maskr, pw_ref[...], 0.0)
    actw = jnp.where(maskr, act * pwcol, 0.0)
    zg_ref[...] += lax.dot_general(actw, wo, (((1,), (1,)), ((), ())),
                                   preferred_element_type=jnp.float32)

    cexc = cex_ref[c] & (_E - 1)
    cexp = cex_ref[jnp.maximum(c - 1, 0)] & (_E - 1)
    firstc = jnp.logical_or(c == 0, cexc != cexp)

    @pl.when(firstc)
    def _():
        rowc = lax.dot_general(gb, wo, (((1,), (1,)), ((), ())),
                               preferred_element_type=jnp.float32)
        const_scr[pl.ds(cexc, 1), :] += rowc

    @pl.when(jnp.logical_and(c == _MAXCH - 1, i == _NI - 1))
    def _():
        const_full = const_scr[...] + bo_ref[...]
        r0 = lax.dot_general(any_ref[0:1, :], const_full, (((1,), (0,)), ((), ())),
                             preferred_element_type=jnp.float32)
        r1 = lax.dot_general(any_ref[1:2, :], const_full, (((1,), (0,)), ((), ())),
                             preferred_element_type=jnp.float32)
        oinit_ref[...] = w0_ref[...] * r0 + w1_ref[...] * r1


def _ffn(cex, val, xg, pw, w0, w1, anyf, Wib, bi4, Wob, bo):
    grid_spec = pltpu.PrefetchScalarGridSpec(
        num_scalar_prefetch=2,
        grid=(_MAXCH, _NI),
        in_specs=[
            pl.BlockSpec((_CH, _D), lambda c, i, cex, val: (c, 0)),
            pl.BlockSpec((_CH, 1), lambda c, i, cex, val: (c, 0)),
            pl.BlockSpec((_T, 1), lambda c, i, cex, val: (0, 0)),
            pl.BlockSpec((_T, 1), lambda c, i, cex, val: (0, 0)),
            pl.BlockSpec((2, _E), lambda c, i, cex, val: (0, 0)),
            pl.BlockSpec((1, _IB, _D), lambda c, i, cex, val: (cex[c] & (_E - 1), i, 0)),
            pl.BlockSpec((1, 1, 1, _IB), lambda c, i, cex, val: (cex[c] & (_E - 1), i, 0, 0)),
            pl.BlockSpec((1, _D, _IB), lambda c, i, cex, val: (cex[c] & (_E - 1), 0, i)),
            pl.BlockSpec((_E, _D), lambda c, i, cex, val: (0, 0)),
        ],
        out_specs=[
            pl.BlockSpec((_CH, _D), lambda c, i, cex, val: (c, 0)),
            pl.BlockSpec((_T, _D), lambda c, i, cex, val: (0, 0)),
        ],
        scratch_shapes=[pltpu.VMEM((_E, _D), jnp.float32)],
    )
    return pl.pallas_call(
        _ffn_body,
        grid_spec=grid_spec,
        out_shape=[
            jax.ShapeDtypeStruct((_PAD, _D), jnp.float32),
            jax.ShapeDtypeStruct((_T, _D), jnp.float32),
        ],
    )(cex, val, xg, pw.reshape(_PAD, 1), w0, w1, anyf, Wib, bi4, Wob, bo)


_TPW = _T // _NW


def _combine_body(zg_hbm, oinit_hbm, inv_hbm, out_hbm, idxv, idxh, rows, basev, sem):
    wid = lax.axis_index("s") * 2 + lax.axis_index("c")
    tbase = wid * _TPW
    pltpu.sync_copy(inv_hbm.at[pl.ds(2 * tbase, 2 * _TPW)], idxv)
    for h in range(2):
        for v in range(4):
            idxh[pl.ds(v * 16, 16)] = (
                idxv[pl.ds(h * 64 + v * 16, 16)] & (_PAD - 1))
        pltpu.async_copy(zg_hbm.at[idxh], rows, sem).wait()
        pltpu.sync_copy(oinit_hbm.at[pl.ds(tbase + h * 32, 32)], basev)

        def body(t, carry):
            for u in range(_D // 16):
                sl = pl.ds(u * 16, 16)
                basev[t, sl] = (basev[t, sl] + rows[2 * t, sl]
                                + rows[2 * t + 1, sl])
            return carry

        lax.fori_loop(0, 32, body, jnp.int32(0))
        pltpu.sync_copy(basev, out_hbm.at[pl.ds(tbase + h * 32, 32)])


def _combine(zg, oinit, inv):
    f = functools.partial(
        pl.kernel,
        mesh=plsc.VectorSubcoreMesh(core_axis_name="c", subcore_axis_name="s"),
        compiler_params=pltpu.CompilerParams(needs_layout_passes=False),
        out_type=jax.ShapeDtypeStruct((_T, _D), jnp.float32),
        scratch_types=[
            pltpu.VMEM((2 * _TPW,), jnp.int32),
            pltpu.VMEM((64,), jnp.int32),
            pltpu.VMEM((64, _D), jnp.float32),
            pltpu.VMEM((32, _D), jnp.float32),
            pltpu.SemaphoreType.DMA,
        ],
    )
    return f(_combine_body)(zg, oinit, inv)


def kernel(x, Wr1, Wr2, Wi, bi, Wo, bo):
    B, T, D = x.shape
    xf = x.reshape(T, D)
    sel2, ws2, w0, w1, anyf = _router(xf, Wr1, Wr2)
    perm, pw, inv, cex, val = _dispatch(sel2.reshape(_NSLOT), ws2.reshape(_NSLOT))
    xg = _gather(perm, xf)
    zg, oinit = _ffn(cex, val, xg, pw, w0, w1, anyf,
                     Wi, bi.reshape(_E, _NI, 1, _IB), Wo, bo)
    out = _combine(zg, oinit, inv)
    return out.reshape(B, T, D)


# speedup vs baseline: 1.0357x; 1.0357x over previous
import functools

import jax
import jax.numpy as jnp
from jax import lax
from jax.experimental import pallas as pl
from jax.experimental.pallas import tpu as pltpu
from jax.experimental.pallas import tpu_sc as plsc

_T = 2048
_D = 768
_I = 3072
_E = 8
_IB = 512
_NI = _I // _IB
_CH = 512
_MAXCH = 16
_NSLOT = 2 * _T
_PAD = _MAXCH * _CH


def _gelu(v):
    return 0.5 * v * (1.0 + lax.erf(v * (2.0 ** -0.5)))


def _router_body(x_ref, wr1_ref, wr2_ref,
                 sel_ref, ws_ref, w0_ref, w1_ref, any_ref):
    x = x_ref[...]
    h = _gelu(lax.dot_general(x, wr1_ref[...], (((1,), (1,)), ((), ())),
                              preferred_element_type=jnp.float32))
    logits = lax.dot_general(h, wr2_ref[...], (((1,), (1,)), ((), ())),
                             preferred_element_type=jnp.float32)
    m = jnp.max(logits, axis=-1, keepdims=True)
    ex = jnp.exp(logits - m)
    p = ex / jnp.sum(ex, axis=-1, keepdims=True)

    iota = lax.broadcasted_iota(jnp.int32, (_T, _E), 1)
    big = jnp.int32(_E + 1)
    m0 = jnp.max(p, axis=-1, keepdims=True)
    i0 = jnp.min(jnp.where(p == m0, iota, big), axis=-1, keepdims=True)
    p2 = jnp.where(iota == i0, -1.0, p)
    m1 = jnp.max(p2, axis=-1, keepdims=True)
    i1 = jnp.min(jnp.where(p2 == m1, iota, big), axis=-1, keepdims=True)

    sel_ref[...] = jnp.concatenate([i0, i1], axis=1)
    ws_ref[...] = jnp.concatenate([m0, m1], axis=1)
    w0_ref[...] = m0
    w1_ref[...] = m1
    oh0 = (iota == i0).astype(jnp.float32)
    oh1 = (iota == i1).astype(jnp.float32)
    any_ref[0:1, :] = jnp.max(oh0, axis=0, keepdims=True)
    any_ref[1:2, :] = jnp.max(oh1, axis=0, keepdims=True)


def _router(xf, Wr1, Wr2):
    return pl.pallas_call(
        _router_body,
        grid=(1,),
        in_specs=[
            pl.BlockSpec((_T, _D), lambda i: (0, 0)),
            pl.BlockSpec((_D // 2, _D), lambda i: (0, 0)),
            pl.BlockSpec((_E, _D // 2), lambda i: (0, 0)),
        ],
        out_specs=[
            pl.BlockSpec((_T, 2), lambda i: (0, 0)),
            pl.BlockSpec((_T, 2), lambda i: (0, 0)),
            pl.BlockSpec((_T, 1), lambda i: (0, 0)),
            pl.BlockSpec((_T, 1), lambda i: (0, 0)),
            pl.BlockSpec((2, _E), lambda i: (0, 0)),
        ],
        out_shape=[
            jax.ShapeDtypeStruct((_T, 2), jnp.int32),
            jax.ShapeDtypeStruct((_T, 2), jnp.float32),
            jax.ShapeDtypeStruct((_T, 1), jnp.float32),
            jax.ShapeDtypeStruct((_T, 1), jnp.float32),
            jax.ShapeDtypeStruct((2, _E), jnp.float32),
        ],
    )(xf, Wr1, Wr2)


_NTILE = 16
_SPT = _NSLOT // _NTILE
_NV = _SPT // 16


def _dispatch_body(sel_hbm, ws_hbm, perm_out, pw_out, inv_out, cex_out, val_out,
                   selv, wsv, cntv, callv, tokv, invv, auxv, zi, zf,
                   csh, psh, wsh, sem):
    cid = lax.axis_index("c")
    tid = lax.axis_index("s")
    iota = lax.iota(jnp.int32, 16)

    @pl.when(cid == 0)
    def _():
        base = tid * _SPT
        sbase = tid * (_PAD // _NTILE)
        pltpu.sync_copy(sel_hbm.at[pl.ds(base, _SPT)], selv)
        pltpu.sync_copy(ws_hbm.at[pl.ds(base, _SPT)], wsv)

        for v in range((_PAD // _NTILE) // 16):
            zi[pl.ds(v * 16, 16)] = jnp.zeros((16,), jnp.int32)
            zf[pl.ds(v * 16, 16)] = jnp.zeros((16,), jnp.float32)
        pltpu.sync_copy(zi, psh.at[pl.ds(sbase, _PAD // _NTILE)])
        pltpu.sync_copy(zf, wsh.at[pl.ds(sbase, _PAD // _NTILE)])

        counts = jnp.zeros((16,), jnp.int32)
        for v in range(_NV):
            sv = selv[pl.ds(v * 16, 16)]
            for e in range(_E):
                ind = jnp.where(sv == e, 1, 0)
                cnt = jnp.sum(ind)
                counts = counts + jnp.where(iota == e, cnt, 0)
        cntv[...] = counts
        pltpu.sync_copy(cntv, csh.at[tid])
        plsc.subcore_barrier()

        pltpu.sync_copy(csh, callv)
        totals = jnp.zeros((16,), jnp.int32)
        tprefix = jnp.zeros((16,), jnp.int32)
        for w in range(_NTILE):
            row = callv[w]
            totals = totals + row
            tprefix = tprefix + jnp.where(jnp.int32(w) < tid, row, 0)
        nch = (totals + (_CH - 1)) >> 9
        incl = plsc.cumsum(nch)
        excl = incl - nch
        basev = excl * _CH
        r_vec = basev + tprefix

        for v in range(_NV):
            sv = selv[pl.ds(v * 16, 16)]
            slot_v = base + v * 16 + iota
            tok_v = slot_v >> 1
            dst_v = jnp.zeros((16,), jnp.int32)
            for e in range(_E):
                m = sv == e
                ind = jnp.where(m, 1, 0)
                pre_rank = plsc.cumsum(ind) - ind
                re = jnp.sum(jnp.where(iota == e, r_vec, 0))
                dst_v = dst_v + jnp.where(m, re + pre_rank, 0)
                r_vec = r_vec + jnp.where(iota == e, jnp.sum(ind), 0)
            tokv[pl.ds(v * 16, 16)] = tok_v
            invv[pl.ds(v * 16, 16)] = dst_v & (_PAD - 1)
        pltpu.sync_copy(tokv, psh.at[invv])
        pltpu.sync_copy(wsv, wsh.at[invv])
        pltpu.sync_copy(invv, inv_out.at[pl.ds(base, _SPT)])
        plsc.subcore_barrier()
        pltpu.sync_copy(psh.at[pl.ds(sbase, _PAD // _NTILE)],
                        perm_out.at[pl.ds(sbase, _PAD // _NTILE)])
        pltpu.sync_copy(wsh.at[pl.ds(sbase, _PAD // _NTILE)],
                        pw_out.at[pl.ds(sbase, _PAD // _NTILE)])

        @pl.when(tid == 0)
        def _():
            total_chunks = jnp.sum(nch)
            cex = jnp.zeros((16,), jnp.int32)
            val = jnp.zeros((16,), jnp.int32)
            for e in range(_E):
                e_excl = jnp.sum(jnp.where(iota == e, excl, 0))
                e_incl = jnp.sum(jnp.where(iota == e, incl, 0))
                e_tot = jnp.sum(jnp.where(iota == e, totals, 0))
                in_r = (iota >= e_excl) & (iota < e_incl)
                cex = cex + jnp.where(in_r, e, 0)
                raw = e_tot - (iota - e_excl) * _CH
                raw = jnp.minimum(jnp.maximum(raw, 0), _CH)
                val = val + jnp.where(in_r, raw, 0)
            cex = jnp.where(iota < total_chunks, cex, _E - 1)
            auxv[...] = cex
            pltpu.sync_copy(auxv, cex_out)
            auxv[...] = val
            pltpu.sync_copy(auxv, val_out)


def _dispatch(sel_flat, ws_flat):
    f = functools.partial(
        pl.kernel,
        mesh=plsc.VectorSubcoreMesh(core_axis_name="c", subcore_axis_name="s"),
        compiler_params=pltpu.CompilerParams(needs_layout_passes=False),
        out_type=[
            jax.ShapeDtypeStruct((_PAD,), jnp.int32),
            jax.ShapeDtypeStruct((_PAD,), jnp.float32),
            jax.ShapeDtypeStruct((_NSLOT,), jnp.int32),
            jax.ShapeDtypeStruct((16,), jnp.int32),
            jax.ShapeDtypeStruct((16,), jnp.int32),
        ],
        scratch_types=[
            pltpu.VMEM((_SPT,), jnp.int32),
            pltpu.VMEM((_SPT,), jnp.float32),
            pltpu.VMEM((16,), jnp.int32),
            pltpu.VMEM((_NTILE, 16), jnp.int32),
            pltpu.VMEM((_SPT,), jnp.int32),
            pltpu.VMEM((_SPT,), jnp.int32),
            pltpu.VMEM((16,), jnp.int32),
            pltpu.VMEM((_PAD // _NTILE,), jnp.int32),
            pltpu.VMEM((_PAD // _NTILE,), jnp.float32),
            pltpu.VMEM_SHARED((_NTILE, 16), jnp.int32),
            pltpu.VMEM_SHARED((_PAD,), jnp.int32),
            pltpu.VMEM_SHARED((_PAD,), jnp.float32),
            pltpu.SemaphoreType.DMA,
        ],
    )
    return f(_dispatch_body)(sel_flat, ws_flat)


_NW = 32
_RPW = _PAD // _NW


def _gather_body(perm_hbm, x_hbm, xg_out, idxv, idx2, rows, sem):
    wid = lax.axis_index("s") * 2 + lax.axis_index("c")
    base = wid * _RPW
    pltpu.sync_copy(perm_hbm.at[pl.ds(base, _RPW)], idxv)
    for h in range(2):
        for v in range(_RPW // 32):
            idx2[pl.ds(v * 16, 16)] = (
                idxv[pl.ds(h * (_RPW // 2) + v * 16, 16)] & (_T - 1))
        pltpu.async_copy(x_hbm.at[idx2], rows, sem).wait()
        pltpu.sync_copy(rows, xg_out.at[pl.ds(base + h * (_RPW // 2), _RPW // 2)])


def _gather(perm, xf):
    f = functools.partial(
        pl.kernel,
        mesh=plsc.VectorSubcoreMesh(core_axis_name="c", subcore_axis_name="s"),
        compiler_params=pltpu.CompilerParams(needs_layout_passes=False),
        out_type=jax.ShapeDtypeStruct((_PAD, _D), jnp.float32),
        scratch_types=[
            pltpu.VMEM((_RPW,), jnp.int32),
            pltpu.VMEM((_RPW // 2,), jnp.int32),
            pltpu.VMEM((_RPW // 2, _D), jnp.float32),
            pltpu.SemaphoreType.DMA,
        ],
    )
    return f(_gather_body)(perm, xf)


def _ffn_body(cex_ref, val_ref,
              xg_ref, pw_ref, w0_ref, w1_ref, any_ref, wi_ref, bi_ref, wo_ref,
              bo_ref, zg_ref, oinit_ref, const_scr):
    c = pl.program_id(0)
    i = pl.program_id(1)

    @pl.when(jnp.logical_and(c == 0, i == 0))
    def _():
        const_scr[...] = jnp.zeros((_E, _D), jnp.float32)
        oinit_ref[...] = jnp.zeros((_T, _D), jnp.float32)

    @pl.when(i == 0)
    def _():
        zg_ref[...] = jnp.zeros((_CH, _D), jnp.float32)

    valc = val_ref[c]

    @pl.when(valc > 0)
    def _():
        xg = xg_ref[...]
        wi = wi_ref[0]
        wo = wo_ref[0]
        bi_row = bi_ref[0, 0]
        pre = lax.dot_general(xg, wi, (((1,), (1,)), ((), ())),
                              preferred_element_type=jnp.float32) + bi_row
        act = _gelu(pre) - _gelu(bi_row)
        rowid = lax.broadcasted_iota(jnp.int32, (_CH, 1), 0)
        maskr = rowid < valc
        pwcol = jnp.where(maskr, pw_ref[...], 0.0)
        actw = jnp.where(maskr, act * pwcol, 0.0)
        zg_ref[...] += lax.dot_general(actw, wo, (((1,), (1,)), ((), ())),
                                       preferred_element_type=jnp.float32)

    cexc = cex_ref[c] & (_E - 1)
    cexp = cex_ref[jnp.maximum(c - 1, 0)] & (_E - 1)
    firstc = jnp.logical_or(c == 0, cexc != cexp)

    @pl.when(firstc)
    def _():
        gb2 = _gelu(bi_ref[0, 0])
        rowc = lax.dot_general(gb2, wo_ref[0], (((1,), (1,)), ((), ())),
                               preferred_element_type=jnp.float32)
        const_scr[pl.ds(cexc, 1), :] += rowc

    @pl.when(jnp.logical_and(c == _MAXCH - 1, i == _NI - 1))
    def _():
        const_full = const_scr[...] + bo_ref[...]
        r0 = lax.dot_general(any_ref[0:1, :], const_full, (((1,), (0,)), ((), ())),
                             preferred_element_type=jnp.float32)
        r1 = lax.dot_general(any_ref[1:2, :], const_full, (((1,), (0,)), ((), ())),
                             preferred_element_type=jnp.float32)
        oinit_ref[...] = w0_ref[...] * r0 + w1_ref[...] * r1


def _ffn(cex, val, xg, pw, w0, w1, anyf, Wib, bi4, Wob, bo):
    grid_spec = pltpu.PrefetchScalarGridSpec(
        num_scalar_prefetch=2,
        grid=(_MAXCH, _NI),
        in_specs=[
            pl.BlockSpec((_CH, _D), lambda c, i, cex, val: (c, 0)),
            pl.BlockSpec((_CH, 1), lambda c, i, cex, val: (c, 0)),
            pl.BlockSpec((_T, 1), lambda c, i, cex, val: (0, 0)),
            pl.BlockSpec((_T, 1), lambda c, i, cex, val: (0, 0)),
            pl.BlockSpec((2, _E), lambda c, i, cex, val: (0, 0)),
            pl.BlockSpec((1, _IB, _D), lambda c, i, cex, val: (cex[c] & (_E - 1), i, 0)),
            pl.BlockSpec((1, 1, 1, _IB), lambda c, i, cex, val: (cex[c] & (_E - 1), i, 0, 0)),
            pl.BlockSpec((1, _D, _IB), lambda c, i, cex, val: (cex[c] & (_E - 1), 0, i)),
            pl.BlockSpec((_E, _D), lambda c, i, cex, val: (0, 0)),
        ],
        out_specs=[
            pl.BlockSpec((_CH, _D), lambda c, i, cex, val: (c, 0)),
            pl.BlockSpec((_T, _D), lambda c, i, cex, val: (0, 0)),
        ],
        scratch_shapes=[pltpu.VMEM((_E, _D), jnp.float32)],
    )
    return pl.pallas_call(
        _ffn_body,
        grid_spec=grid_spec,
        out_shape=[
            jax.ShapeDtypeStruct((_PAD, _D), jnp.float32),
            jax.ShapeDtypeStruct((_T, _D), jnp.float32),
        ],
    )(cex, val, xg, pw.reshape(_PAD, 1), w0, w1, anyf, Wib, bi4, Wob, bo)


_TPW = _T // _NW


def _combine_body(zg_hbm, oinit_hbm, inv_hbm, out_hbm, idxv, idxh, rows, basev, sem):
    wid = lax.axis_index("s") * 2 + lax.axis_index("c")
    tbase = wid * _TPW
    pltpu.sync_copy(inv_hbm.at[pl.ds(2 * tbase, 2 * _TPW)], idxv)
    for h in range(2):
        for v in range(4):
            idxh[pl.ds(v * 16, 16)] = (
                idxv[pl.ds(h * 64 + v * 16, 16)] & (_PAD - 1))
        pltpu.async_copy(zg_hbm.at[idxh], rows, sem).wait()
        pltpu.sync_copy(oinit_hbm.at[pl.ds(tbase + h * 32, 32)], basev)

        def body(t, carry):
            for u in range(_D // 16):
                sl = pl.ds(u * 16, 16)
                basev[t, sl] = (basev[t, sl] + rows[2 * t, sl]
                                + rows[2 * t + 1, sl])
            return carry

        lax.fori_loop(0, 32, body, jnp.int32(0))
        pltpu.sync_copy(basev, out_hbm.at[pl.ds(tbase + h * 32, 32)])


def _combine(zg, oinit, inv):
    f = functools.partial(
        pl.kernel,
        mesh=plsc.VectorSubcoreMesh(core_axis_name="c", subcore_axis_name="s"),
        compiler_params=pltpu.CompilerParams(needs_layout_passes=False),
        out_type=jax.ShapeDtypeStruct((_T, _D), jnp.float32),
        scratch_types=[
            pltpu.VMEM((2 * _TPW,), jnp.int32),
            pltpu.VMEM((64,), jnp.int32),
            pltpu.VMEM((64, _D), jnp.float32),
            pltpu.VMEM((32, _D), jnp.float32),
            pltpu.SemaphoreType.DMA,
        ],
    )
    return f(_combine_body)(zg, oinit, inv)


def kernel(x, Wr1, Wr2, Wi, bi, Wo, bo):
    B, T, D = x.shape
    xf = x.reshape(T, D)
    sel2, ws2, w0, w1, anyf = _router(xf, Wr1, Wr2)
    perm, pw, inv, cex, val = _dispatch(sel2.reshape(_NSLOT), ws2.reshape(_NSLOT))
    xg = _gather(perm, xf)
    zg, oinit = _ffn(cex, val, xg, pw, w0, w1, anyf,
                     Wi, bi.reshape(_E, _NI, 1, _IB), Wo, bo)
    out = _combine(zg, oinit, inv)
    return out.reshape(B, T, D)


# speedup vs baseline: 2.2290x; 2.1523x over previous
import jax
import jax.numpy as jnp
from jax import lax
from jax.experimental import pallas as pl
from jax.experimental.pallas import tpu as pltpu

_T = 2048
_D = 768
_I = 3072
_E = 8
_IB = 1024
_NI = _I // _IB


def _gelu(v):
    return 0.5 * v * (1.0 + lax.erf(v * (2.0 ** -0.5)))


def _moe_body(x_ref, wr1_ref, wr2_ref, wi_ref, bi_ref, wo_ref, bo_ref,
              out_ref, c_scr, w0_scr, w1_scr, any_scr, const_scr):
    e = pl.program_id(0)
    i = pl.program_id(1)

    @pl.when(jnp.logical_and(e == 0, i == 0))
    def _router():
        x = x_ref[...]
        h = _gelu(lax.dot_general(x, wr1_ref[...], (((1,), (1,)), ((), ())),
                                  preferred_element_type=jnp.float32))
        logits = lax.dot_general(h, wr2_ref[...], (((1,), (1,)), ((), ())),
                                 preferred_element_type=jnp.float32)
        m = jnp.max(logits, axis=-1, keepdims=True)
        ex = jnp.exp(logits - m)
        p = ex / jnp.sum(ex, axis=-1, keepdims=True)

        iota = lax.broadcasted_iota(jnp.int32, (_T, _E), 1)
        big = jnp.int32(_E + 1)
        m0 = jnp.max(p, axis=-1, keepdims=True)
        i0 = jnp.min(jnp.where(p == m0, iota, big), axis=-1, keepdims=True)
        oh0 = (iota == i0).astype(jnp.float32)
        p2 = jnp.where(iota == i0, -1.0, p)
        m1 = jnp.max(p2, axis=-1, keepdims=True)
        i1 = jnp.min(jnp.where(p2 == m1, iota, big), axis=-1, keepdims=True)
        oh1 = (iota == i1).astype(jnp.float32)

        c_scr[...] = m0 * oh0 + m1 * oh1
        w0_scr[...] = m0
        w1_scr[...] = m1
        any_scr[0:1, :] = jnp.max(oh0, axis=0, keepdims=True)
        any_scr[1:2, :] = jnp.max(oh1, axis=0, keepdims=True)
        const_scr[...] = jnp.zeros((_E, _D), jnp.float32)
        out_ref[...] = jnp.zeros((_T, _D), jnp.float32)

    x = x_ref[...]
    wi = wi_ref[0]
    wo = wo_ref[0]
    bi_row = bi_ref[0, 0]

    pre = lax.dot_general(x, wi, (((1,), (1,)), ((), ())),
                          preferred_element_type=jnp.float32) + bi_row
    act = _gelu(pre)
    iota = lax.broadcasted_iota(jnp.int32, (_T, _E), 1)
    ce = jnp.sum(jnp.where(iota == e, c_scr[...], 0.0), axis=-1, keepdims=True)
    out_ref[...] += lax.dot_general(act * ce, wo, (((1,), (1,)), ((), ())),
                                    preferred_element_type=jnp.float32)

    g = _gelu(bi_row)
    rowc = lax.dot_general(g, wo, (((1,), (1,)), ((), ())),
                           preferred_element_type=jnp.float32)
    const_scr[pl.ds(e, 1), :] += rowc

    @pl.when(jnp.logical_and(e == _E - 1, i == _NI - 1))
    def _correction():
        constmm = const_scr[...]
        const_full = constmm + bo_ref[...]
        r0 = lax.dot_general(any_scr[0:1, :], const_full, (((1,), (0,)), ((), ())),
                             preferred_element_type=jnp.float32)
        r1 = lax.dot_general(any_scr[1:2, :], const_full, (((1,), (0,)), ((), ())),
                             preferred_element_type=jnp.float32)
        corr = lax.dot_general(c_scr[...], constmm, (((1,), (0,)), ((), ())),
                               preferred_element_type=jnp.float32)
        out_ref[...] += w0_scr[...] * r0 + w1_scr[...] * r1 - corr


def kernel(x, Wr1, Wr2, Wi, bi, Wo, bo):
    B, T, D = x.shape
    xf = x.reshape(T, D)
    out = pl.pallas_call(
        _moe_body,
        grid=(_E, _NI),
        in_specs=[
            pl.BlockSpec((_T, _D), lambda e, i: (0, 0)),
            pl.BlockSpec((_D // 2, _D), lambda e, i: (0, 0)),
            pl.BlockSpec((_E, _D // 2), lambda e, i: (0, 0)),
            pl.BlockSpec((1, _IB, _D), lambda e, i: (e, i, 0)),
            pl.BlockSpec((1, 1, 1, _IB), lambda e, i: (e, i, 0, 0)),
            pl.BlockSpec((1, _D, _IB), lambda e, i: (e, 0, i)),
            pl.BlockSpec((_E, _D), lambda e, i: (0, 0)),
        ],
        out_specs=pl.BlockSpec((_T, _D), lambda e, i: (0, 0)),
        out_shape=jax.ShapeDtypeStruct((T, D), jnp.float32),
        scratch_shapes=[
            pltpu.VMEM((_T, _E), jnp.float32),
            pltpu.VMEM((_T, 1), jnp.float32),
            pltpu.VMEM((_T, 1), jnp.float32),
            pltpu.VMEM((2, _E), jnp.float32),
            pltpu.VMEM((_E, _D), jnp.float32),
        ],
    )(xf, Wr1, Wr2, Wi, bi.reshape(_E, _NI, 1, _IB), Wo, bo)
    return out.reshape(B, T, D)
